# baseline (speedup 1.0000x reference)
import functools

import jax
import jax.numpy as jnp
from jax import lax
from jax.experimental import pallas as pl
from jax.experimental.pallas import tpu as pltpu
from jax.experimental.pallas import tpu_sc as plsc

N = 10000
E = 320000
R = 8
B = 8
IN = 128
HID = 256
OUT = 128

_NC = 2
_NS = 16
_L = 16


def _make_edge_pass(n_tab: int, d: int, edge_split: bool):
    K = 80
    n_workers = _NS * _NC if edge_split else _NS
    e_per_w = E // n_workers
    NR = e_per_w // K
    NPAIR = NR // 2
    ZR = 80
    ET = 2000
    n_row_chunks = N // ZR
    row_rounds = -(-n_row_chunks // _NS)
    assert e_per_w % K == 0 and N % ZR == 0 and e_per_w % ET == 0

    mesh = plsc.VectorSubcoreMesh(core_axis_name="c", subcore_axis_name="s",
                                  num_cores=_NC, num_subcores=_NS)

    @functools.partial(
        pl.kernel,
        out_type=jax.ShapeDtypeStruct((_NC, N, d), jnp.float32),
        mesh=mesh,
        scratch_types=[
            pltpu.VMEM_SHARED((N, d), jnp.float32),
            pltpu.VMEM((e_per_w,), jnp.int32),
            pltpu.VMEM((ET,), jnp.int32),
            pltpu.VMEM((K,), jnp.int32),
            pltpu.VMEM((K,), jnp.int32),
            pltpu.VMEM((K, d), jnp.float32),
            pltpu.VMEM((K, d), jnp.float32),
            pltpu.SemaphoreType.DMA,
            pltpu.SemaphoreType.DMA,
        ],
    )
    def edge_pass(tab, src, et, dst, out, acc, srcv, etv, dstA, dstB,
                  rowsA, rowsB, sem, sem2):
        c = lax.axis_index("c")
        s = lax.axis_index("s")

        if edge_split:
            half = 0
            widx = s * _NC + c
        else:
            half = c * n_tab
            widx = s
        w0 = widx * e_per_w

        pltpu.sync_copy(src.at[pl.ds(w0, e_per_w)], srcv)

        groups = d // _L

        def _zfill(t, carry):
            rowsA[t // groups, pl.ds((t % groups) * _L, _L)] = jnp.zeros((_L,), jnp.float32)
            return carry
        lax.fori_loop(0, K * groups, _zfill, 0)

        for q in range(row_rounds):
            chunk_id = s + q * _NS
            @pl.when(chunk_id < n_row_chunks)
            def _():
                pltpu.sync_copy(rowsA, acc.at[pl.ds(chunk_id * ZR, ZR)])

        def _etblk(b, carry):
            pltpu.sync_copy(et.at[pl.ds(w0 + b * ET, ET)], etv)

            def _gidx(t, carry2):
                sl = pl.ds(b * ET + t * _L, _L)
                srcv[sl] = half + etv[pl.ds(t * _L, _L)] * N + srcv[sl]
                return carry2
            lax.fori_loop(0, ET // _L, _gidx, 0)
            return carry
        lax.fori_loop(0, e_per_w // ET, _etblk, 0)

        def _gather_start(j, buf):
            return pltpu.async_copy(tab.at[pl.ds(0, K)], buf, sem)

        def _dst_start(j, buf):
            return pltpu.async_copy(dst.at[pl.ds(w0 + j * K, K)], buf, sem2)

        d0 = _gather_start(0, rowsA)
        e0 = _dst_start(0, dstA)
        plsc.subcore_barrier()
        d0.wait()
        e0.wait()

        def _scatter(buf, dbuf):
            pass

        def _pair(t, carry):
            a = 2 * t
            db = _gather_start(a + 1, rowsB)
            eb = _dst_start(a + 1, dstB)
            _scatter(rowsA, dstA)
            db.wait()
            eb.wait()

            @pl.when(a + 2 < NR)
            def _():
                da = _gather_start(a + 2, rowsA)
                ea = _dst_start(a + 2, dstA)
                _scatter(rowsB, dstB)
                da.wait()
                ea.wait()

            @pl.when(a + 2 >= NR)
            def _():
                _scatter(rowsB, dstB)
            return carry
        lax.fori_loop(0, NPAIR, _pair, 0)
        if NR % 2:
            _scatter(rowsA, dstA)

        plsc.subcore_barrier()
        for q in range(row_rounds):
            chunk_id = s + q * _NS
            @pl.when(chunk_id < n_row_chunks)
            def _():
                r0 = chunk_id * ZR
                pltpu.sync_copy(acc.at[pl.ds(r0, ZR)], out.at[c, pl.ds(r0, ZR)])

    return edge_pass


def _make_wcomb(n_in: int, n_out: int):
    def body(c_ref, b_ref, o_ref):
        o_ref[...] = jnp.dot(c_ref[...], b_ref[...],
                             preferred_element_type=jnp.float32)
    return pl.pallas_call(
        body,
        grid=(1,),
        in_specs=[
            pl.BlockSpec((R, B), lambda i: (0, 0)),
            pl.BlockSpec((B, n_in * n_out), lambda i: (0, 0)),
        ],
        out_specs=pl.BlockSpec((R, n_in * n_out), lambda i: (0, 0)),
        out_shape=jax.ShapeDtypeStruct((R, n_in * n_out), jnp.float32),
    )


def _make_proj(n_in: int, dh: int, bn: int, nh: int):
    nb = N // bn

    def body(x_ref, w_ref, o_ref):
        o_ref[0, 0] = jnp.dot(x_ref[...], w_ref[0, 0],
                              preferred_element_type=jnp.float32)
    return pl.pallas_call(
        body,
        grid=(nb, R, nh),
        in_specs=[
            pl.BlockSpec((bn, n_in), lambda i, r, h: (i, 0)),
            pl.BlockSpec((1, 1, n_in, dh), lambda i, r, h: (h, r, 0, 0)),
        ],
        out_specs=pl.BlockSpec((1, 1, bn, dh), lambda i, r, h: (h, r, i, 0)),
        out_shape=jax.ShapeDtypeStruct((nh, R, N, dh), jnp.float32),
    )


def _make_combine(n_in: int, n_out: int, relu: bool, bn: int, cat: bool):
    dh = n_out // 2 if cat else n_out
    nb = N // bn

    def body(lo_ref, hi_ref, x_ref, w_ref, b_ref, o_ref):
        m = jnp.dot(x_ref[...], w_ref[...], preferred_element_type=jnp.float32)
        if cat:
            agg = jnp.concatenate([lo_ref[0], hi_ref[0]], axis=-1)
        else:
            agg = lo_ref[0] + hi_ref[0]
        res = agg + m + b_ref[...]
        o_ref[...] = jnp.maximum(res, 0.0) if relu else res
    return pl.pallas_call(
        body,
        grid=(nb,),
        in_specs=[
            pl.BlockSpec((1, bn, dh), lambda i: (0, i, 0)),
            pl.BlockSpec((1, bn, dh), lambda i: (1, i, 0)),
            pl.BlockSpec((bn, n_in), lambda i: (i, 0)),
            pl.BlockSpec((n_in, n_out), lambda i: (0, 0)),
            pl.BlockSpec((1, n_out), lambda i: (0, 0)),
        ],
        out_specs=pl.BlockSpec((bn, n_out), lambda i: (i, 0)),
        out_shape=jax.ShapeDtypeStruct((N, n_out), jnp.float32),
    )


_wcomb1 = _make_wcomb(IN, HID)
_wcomb2 = _make_wcomb(HID, OUT)
_proj1 = _make_proj(IN, HID // 2, 2000, 2)
_proj2 = _make_proj(HID, OUT, 2000, 1)
_edge1 = _make_edge_pass(R * N, HID // 2, edge_split=False)
_edge2 = _make_edge_pass(R * N, OUT, edge_split=True)
_comb1 = _make_combine(IN, HID, True, 2000, cat=True)
_comb2 = _make_combine(HID, OUT, False, 2000, cat=False)


def kernel(x, edge_index, etype, bases1, coeff1, loop1, bias1,
           bases2, coeff2, loop2, bias2):
    src = edge_index[0]
    dst = edge_index[1]

    w1 = _wcomb1(coeff1, bases1.reshape(B, IN * HID)).reshape(R, IN, HID)
    w1s = jnp.stack([w1[..., :HID // 2], w1[..., HID // 2:]])
    p1 = _proj1(x, w1s)
    agg1 = _edge1(p1.reshape(_NC * R * N, HID // 2), src, etype, dst)
    h = _comb1(agg1, agg1, x, loop1, bias1.reshape(1, HID))

    w2 = _wcomb2(coeff2, bases2.reshape(B, HID * OUT)).reshape(1, R, HID, OUT)
    p2 = _proj2(h, w2)
    agg2 = _edge2(p2.reshape(R * N, OUT), src, etype, dst)
    out = _comb2(agg2, agg2, h, loop2, bias2.reshape(1, OUT))
    return out


# speedup vs baseline: 2.6777x; 2.6777x over previous
import functools

import jax
import jax.numpy as jnp
from jax import lax
from jax.experimental import pallas as pl
from jax.experimental.pallas import tpu as pltpu
from jax.experimental.pallas import tpu_sc as plsc

N = 10000
E = 320000
R = 8
B = 8
IN = 128
HID = 256
OUT = 128

_NC = 2
_NS = 16
_L = 16


def _make_edge_pass(n_tab: int, d: int, edge_split: bool):
    K = 80
    n_workers = _NS * _NC if edge_split else _NS
    e_per_w = E // n_workers
    NR = e_per_w // K
    NSB = 5
    SBR = NR // NSB
    SB_E = SBR * K
    ZR = 80
    n_row_chunks = N // ZR
    row_rounds = -(-n_row_chunks // _NS)
    assert e_per_w % K == 0 and N % ZR == 0 and NR % NSB == 0 and SBR > 9

    mesh = plsc.VectorSubcoreMesh(core_axis_name="c", subcore_axis_name="s",
                                  num_cores=_NC, num_subcores=_NS)

    @functools.partial(
        pl.kernel,
        out_type=jax.ShapeDtypeStruct((_NC, N, d), jnp.float32),
        mesh=mesh,
    scratch_types=[
            pltpu.VMEM_SHARED((N, d), jnp.float32),
            pltpu.VMEM((2 * SB_E,), jnp.int32),
            pltpu.VMEM((SB_E,), jnp.int32),
            pltpu.VMEM((K,), jnp.int32),
            pltpu.VMEM((K,), jnp.int32),
            pltpu.VMEM((K,), jnp.int32),
            pltpu.VMEM((K, d), jnp.float32),
            pltpu.VMEM((K, d), jnp.float32),
            pltpu.VMEM((K, d), jnp.float32),
            pltpu.SemaphoreType.DMA,
            pltpu.SemaphoreType.DMA,
            pltpu.SemaphoreType.DMA,
            pltpu.SemaphoreType.DMA,
            pltpu.SemaphoreType.DMA,
            pltpu.SemaphoreType.DMA,
            pltpu.SemaphoreType.DMA,
        ],
    )
    def edge_pass(tab, src, et, dst, out, acc, S, etv, D0, D1, D2,
                  R0, R1, R2, g0, g1, g2, e0, e1, e2, stsem):
        c = lax.axis_index("c")
        s = lax.axis_index("s")

        if edge_split:
            half = 0
            widx = s * _NC + c
        else:
            half = c * n_tab
            widx = s
        w0 = widx * e_per_w

        RB = (R0, R1, R2)
        DB = (D0, D1, D2)
        GS = (g0, g1, g2)
        ES = (e0, e1, e2)

        groups = d // _L

        def _zfill(t, carry):
            R0[t // groups, pl.ds((t % groups) * _L, _L)] = jnp.zeros((_L,), jnp.float32)
            return carry
        lax.fori_loop(0, K * groups, _zfill, 0)

        for q in range(row_rounds):
            chunk_id = s + q * _NS
            @pl.when(chunk_id < n_row_chunks)
            def _():
                pltpu.sync_copy(R0, acc.at[pl.ds(chunk_id * ZR, ZR)])

        def _compute_gidx(soff):
            def _gidx(t, carry):
                sl = pl.ds(soff + t * _L, _L)
                S[sl] = half + etv[pl.ds(t * _L, _L)] * N + S[sl]
                return carry
            lax.fori_loop(0, SB_E // _L, _gidx, 0)

        pltpu.sync_copy(src.at[pl.ds(w0, SB_E)], S.at[pl.ds(0, SB_E)])
        pltpu.sync_copy(et.at[pl.ds(w0, SB_E)], etv)
        _compute_gidx(0)

        def _gather_start(j, i):
            off = (j % (2 * SBR)) * K
            pltpu.async_copy(tab.at[S.at[pl.ds(off, K)]], RB[i], GS[i])

        def _gather_wait(i):
            pltpu.make_async_copy(tab.at[S.at[pl.ds(0, K)]], RB[i], GS[i]).wait()

        def _dst_start(j, i):
            pltpu.async_copy(dst.at[pl.ds(w0 + j * K, K)], DB[i], ES[i])

        def _dst_wait(i):
            pltpu.make_async_copy(dst.at[pl.ds(w0, K)], DB[i], ES[i]).wait()

        for i in range(3):
            _gather_start(i, i)
            _dst_start(i, i)
        plsc.subcore_barrier()

        def _body(u, carry):
            for i in range(3):
                r = 3 * u + i
                _gather_wait(i)
                _dst_wait(i)
                pltpu.sync_copy(RB[i], acc.at[DB[i]], add=True)
                nxt = r + 3

                @pl.when(nxt < NR)
                def _():
                    _gather_start(nxt, i)
                    _dst_start(nxt, i)

                sb1 = r // SBR + 1

                @pl.when(jnp.logical_and(r % SBR == 0, sb1 < NSB))
                def _():
                    soff = (sb1 % 2) * SB_E
                    pltpu.async_copy(src.at[pl.ds(w0 + sb1 * SB_E, SB_E)],
                                     S.at[pl.ds(soff, SB_E)], stsem)
                    pltpu.async_copy(et.at[pl.ds(w0 + sb1 * SB_E, SB_E)],
                                     etv, stsem)

                @pl.when(jnp.logical_and(r % SBR == SBR - 6, sb1 < NSB))
                def _():
                    pltpu.make_async_copy(src.at[pl.ds(w0, SB_E)],
                                          S.at[pl.ds(0, SB_E)], stsem).wait()
                    pltpu.make_async_copy(et.at[pl.ds(w0, SB_E)], etv,
                                          stsem).wait()
                    _compute_gidx((sb1 % 2) * SB_E)
            return carry
        lax.fori_loop(0, NR // 3, _body, 0)
        for i in range(NR % 3):
            _gather_wait(i)
            _dst_wait(i)
            pltpu.sync_copy(RB[i], acc.at[DB[i]], add=True)

        plsc.subcore_barrier()
        for q in range(row_rounds):
            chunk_id = s + q * _NS
            @pl.when(chunk_id < n_row_chunks)
            def _():
                r0 = chunk_id * ZR
                pltpu.sync_copy(acc.at[pl.ds(r0, ZR)], out.at[c, pl.ds(r0, ZR)])

    return edge_pass


def _make_wcomb(n_in: int, n_out: int):
    def body(c_ref, b_ref, o_ref):
        o_ref[...] = jnp.dot(c_ref[...], b_ref[...],
                             preferred_element_type=jnp.float32)
    return pl.pallas_call(
        body,
        grid=(1,),
        in_specs=[
            pl.BlockSpec((R, B), lambda i: (0, 0)),
            pl.BlockSpec((B, n_in * n_out), lambda i: (0, 0)),
        ],
        out_specs=pl.BlockSpec((R, n_in * n_out), lambda i: (0, 0)),
        out_shape=jax.ShapeDtypeStruct((R, n_in * n_out), jnp.float32),
    )


def _make_proj(n_in: int, dh: int, bn: int, nh: int):
    nb = N // bn

    def body(x_ref, w_ref, o_ref):
        o_ref[0, 0] = jnp.dot(x_ref[...], w_ref[0, 0],
                              preferred_element_type=jnp.float32)
    return pl.pallas_call(
        body,
        grid=(nb, R, nh),
        in_specs=[
            pl.BlockSpec((bn, n_in), lambda i, r, h: (i, 0)),
            pl.BlockSpec((1, 1, n_in, dh), lambda i, r, h: (h, r, 0, 0)),
        ],
        out_specs=pl.BlockSpec((1, 1, bn, dh), lambda i, r, h: (h, r, i, 0)),
        out_shape=jax.ShapeDtypeStruct((nh, R, N, dh), jnp.float32),
    )


def _make_combine(n_in: int, n_out: int, relu: bool, bn: int, cat: bool):
    dh = n_out // 2 if cat else n_out
    nb = N // bn

    def body(lo_ref, hi_ref, x_ref, w_ref, b_ref, o_ref):
        m = jnp.dot(x_ref[...], w_ref[...], preferred_element_type=jnp.float32)
        if cat:
            agg = jnp.concatenate([lo_ref[0], hi_ref[0]], axis=-1)
        else:
            agg = lo_ref[0] + hi_ref[0]
        res = agg + m + b_ref[...]
        o_ref[...] = jnp.maximum(res, 0.0) if relu else res
    return pl.pallas_call(
        body,
        grid=(nb,),
        in_specs=[
            pl.BlockSpec((1, bn, dh), lambda i: (0, i, 0)),
            pl.BlockSpec((1, bn, dh), lambda i: (1, i, 0)),
            pl.BlockSpec((bn, n_in), lambda i: (i, 0)),
            pl.BlockSpec((n_in, n_out), lambda i: (0, 0)),
            pl.BlockSpec((1, n_out), lambda i: (0, 0)),
        ],
        out_specs=pl.BlockSpec((bn, n_out), lambda i: (i, 0)),
        out_shape=jax.ShapeDtypeStruct((N, n_out), jnp.float32),
    )


_wcomb1 = _make_wcomb(IN, HID)
_wcomb2 = _make_wcomb(HID, OUT)
_proj1 = _make_proj(IN, HID // 2, 2000, 2)
_proj2 = _make_proj(HID, OUT, 2000, 1)
_edge1 = _make_edge_pass(R * N, HID // 2, edge_split=False)
_edge2 = _make_edge_pass(R * N, OUT, edge_split=True)
_comb1 = _make_combine(IN, HID, True, 2000, cat=True)
_comb2 = _make_combine(HID, OUT, False, 2000, cat=False)


def kernel(x, edge_index, etype, bases1, coeff1, loop1, bias1,
           bases2, coeff2, loop2, bias2):
    src = edge_index[0]
    dst = edge_index[1]

    w1 = _wcomb1(coeff1, bases1.reshape(B, IN * HID)).reshape(R, IN, HID)
    w1s = jnp.stack([w1[..., :HID // 2], w1[..., HID // 2:]])
    p1 = _proj1(x, w1s)
    agg1 = _edge1(p1.reshape(_NC * R * N, HID // 2), src, etype, dst)
    h = _comb1(agg1, agg1, x, loop1, bias1.reshape(1, HID))

    w2 = _wcomb2(coeff2, bases2.reshape(B, HID * OUT)).reshape(1, R, HID, OUT)
    p2 = _proj2(h, w2)
    agg2 = _edge2(p2.reshape(R * N, OUT), src, etype, dst)
    out = _comb2(agg2, agg2, h, loop2, bias2.reshape(1, OUT))
    return out


# speedup vs baseline: 2.8590x; 1.0677x over previous
import functools

import jax
import jax.numpy as jnp
from jax import lax
from jax.experimental import pallas as pl
from jax.experimental.pallas import tpu as pltpu
from jax.experimental.pallas import tpu_sc as plsc

N = 10000
E = 320000
R = 8
B = 8
IN = 128
HID = 256
OUT = 128

_NC = 2
_NS = 16
_L = 16


def _make_edge_pass(n_tab: int, d: int, edge_split: bool):
    K = 80
    n_workers = _NS * _NC if edge_split else _NS
    e_per_w = E // n_workers
    NR = e_per_w // K
    NSB = 5
    SBR = NR // NSB
    SB_E = SBR * K
    ZR = 80
    n_row_chunks = N // ZR
    row_rounds = -(-n_row_chunks // _NS)
    assert e_per_w % K == 0 and N % ZR == 0 and NR % NSB == 0 and SBR > 9

    mesh = plsc.VectorSubcoreMesh(core_axis_name="c", subcore_axis_name="s",
                                  num_cores=_NC, num_subcores=_NS)

    @functools.partial(
        pl.kernel,
        out_type=jax.ShapeDtypeStruct((_NC, N, d), jnp.float32),
        mesh=mesh,
    scratch_types=[
            pltpu.VMEM_SHARED((N, d), jnp.float32),
            pltpu.VMEM((2 * SB_E,), jnp.int32),
            pltpu.VMEM((SB_E,), jnp.int32),
            pltpu.VMEM((K,), jnp.int32),
            pltpu.VMEM((K,), jnp.int32),
            pltpu.VMEM((K,), jnp.int32),
            pltpu.VMEM((K, d), jnp.float32),
            pltpu.VMEM((K, d), jnp.float32),
            pltpu.VMEM((K, d), jnp.float32),
            pltpu.SemaphoreType.DMA,
            pltpu.SemaphoreType.DMA,
            pltpu.SemaphoreType.DMA,
            pltpu.SemaphoreType.DMA,
            pltpu.SemaphoreType.DMA,
            pltpu.SemaphoreType.DMA,
            pltpu.SemaphoreType.DMA,
        ],
    )
    def edge_pass(tab, src, et, dst, out, acc, S, etv, D0, D1, D2,
                  R0, R1, R2, g0, g1, g2, e0, e1, e2, stsem):
        c = lax.axis_index("c")
        s = lax.axis_index("s")

        if edge_split:
            half = 0
            widx = s * _NC + c
        else:
            half = c * n_tab
            widx = s
        w0 = widx * e_per_w

        RB = (R0, R1, R2)
        DB = (D0, D1, D2)
        GS = (g0, g1, g2)
        ES = (e0, e1, e2)

        groups = d // _L

        def _zfill(t, carry):
            R0[t // groups, pl.ds((t % groups) * _L, _L)] = jnp.zeros((_L,), jnp.float32)
            return carry
        lax.fori_loop(0, K * groups, _zfill, 0)

        for q in range(row_rounds):
            chunk_id = s + q * _NS
            @pl.when(chunk_id < n_row_chunks)
            def _():
                pltpu.sync_copy(R0, acc.at[pl.ds(chunk_id * ZR, ZR)])

        def _compute_gidx(soff):
            def _gidx(t, carry):
                sl = pl.ds(soff + t * _L, _L)
                S[sl] = half + etv[pl.ds(t * _L, _L)] * N + S[sl]
                return carry
            lax.fori_loop(0, SB_E // _L, _gidx, 0)

        pltpu.sync_copy(src.at[pl.ds(w0, SB_E)], S.at[pl.ds(0, SB_E)])
        pltpu.sync_copy(et.at[pl.ds(w0, SB_E)], etv)
        _compute_gidx(0)

        def _gather_start(j, i):
            off = (j % (2 * SBR)) * K
            pltpu.async_copy(tab.at[S.at[pl.ds(off, K)]], RB[i], GS[i])

        def _gather_wait(i):
            pltpu.make_async_copy(tab.at[S.at[pl.ds(0, K)]], RB[i], GS[i]).wait()

        def _dst_start(j, i):
            pltpu.async_copy(dst.at[pl.ds(w0 + j * K, K)], DB[i], ES[i])

        def _dst_wait(i):
            pltpu.make_async_copy(dst.at[pl.ds(w0, K)], DB[i], ES[i]).wait()

        for i in range(3):
            _gather_start(i, i)
            _dst_start(i, i)
        plsc.subcore_barrier()

        def _body(u, carry):
            for i in range(3):
                r = 3 * u + i
                _gather_wait(i)
                _dst_wait(i)
                pltpu.sync_copy(RB[i], acc.at[DB[i]], add=True)
                nxt = r + 3

                @pl.when(nxt < NR)
                def _():
                    _gather_start(nxt, i)
                    _dst_start(nxt, i)

                sb1 = r // SBR + 1

                @pl.when(jnp.logical_and(r % SBR == 0, sb1 < NSB))
                def _():
                    soff = (sb1 % 2) * SB_E
                    pltpu.async_copy(src.at[pl.ds(w0 + sb1 * SB_E, SB_E)],
                                     S.at[pl.ds(soff, SB_E)], stsem)
                    pltpu.async_copy(et.at[pl.ds(w0 + sb1 * SB_E, SB_E)],
                                     etv, stsem)

                @pl.when(jnp.logical_and(r % SBR == SBR - 6, sb1 < NSB))
                def _():
                    pltpu.make_async_copy(src.at[pl.ds(w0, SB_E)],
                                          S.at[pl.ds(0, SB_E)], stsem).wait()
                    pltpu.make_async_copy(et.at[pl.ds(w0, SB_E)], etv,
                                          stsem).wait()
                    _compute_gidx((sb1 % 2) * SB_E)
            return carry
        lax.fori_loop(0, NR // 3, _body, 0)
        for i in range(NR % 3):
            _gather_wait(i)
            _dst_wait(i)
            pltpu.sync_copy(RB[i], acc.at[DB[i]], add=True)

        plsc.subcore_barrier()
        for q in range(row_rounds):
            chunk_id = s + q * _NS
            @pl.when(chunk_id < n_row_chunks)
            def _():
                r0 = chunk_id * ZR
                pltpu.sync_copy(acc.at[pl.ds(r0, ZR)], out.at[c, pl.ds(r0, ZR)])

    return edge_pass


def _make_wcomb(n_in: int, n_out: int):
    def body(c_ref, b_ref, o_ref):
        o_ref[...] = jnp.dot(c_ref[...], b_ref[...],
                             preferred_element_type=jnp.float32)
    return pl.pallas_call(
        body,
        grid=(1,),
        in_specs=[
            pl.BlockSpec((R, B), lambda i: (0, 0)),
            pl.BlockSpec((B, n_in * n_out), lambda i: (0, 0)),
        ],
        out_specs=pl.BlockSpec((R, n_in * n_out), lambda i: (0, 0)),
        out_shape=jax.ShapeDtypeStruct((R, n_in * n_out), jnp.float32),
    )


def _make_proj(n_in: int, dh: int, bn: int, nh: int):
    nb = N // bn

    def body(x_ref, w_ref, o_ref):
        o_ref[0, 0] = jnp.dot(x_ref[...], w_ref[0, 0],
                              preferred_element_type=jnp.float32)
    return pl.pallas_call(
        body,
        grid=(nb, R, nh),
        in_specs=[
            pl.BlockSpec((bn, n_in), lambda i, r, h: (i, 0)),
            pl.BlockSpec((1, 1, n_in, dh), lambda i, r, h: (h, r, 0, 0)),
        ],
        out_specs=pl.BlockSpec((1, 1, bn, dh), lambda i, r, h: (h, r, i, 0)),
        out_shape=jax.ShapeDtypeStruct((nh, R, N, dh), jnp.float32),
    )


def _make_comb_proj(bn: int):
    nb = N // bn

    def body(lo_ref, hi_ref, x_ref, w_ref, b_ref, w2_ref, h_ref, p2_ref):
        m = jnp.dot(x_ref[...], w_ref[...], preferred_element_type=jnp.float32)
        agg = jnp.concatenate([lo_ref[0], hi_ref[0]], axis=-1)
        hblk = jnp.maximum(agg + m + b_ref[...], 0.0)
        h_ref[...] = hblk
        for r in range(R):
            p2_ref[0, r] = jnp.dot(hblk, w2_ref[0, r],
                                   preferred_element_type=jnp.float32)
    return pl.pallas_call(
        body,
        grid=(nb,),
        in_specs=[
            pl.BlockSpec((1, bn, HID // 2), lambda i: (0, i, 0)),
            pl.BlockSpec((1, bn, HID // 2), lambda i: (1, i, 0)),
            pl.BlockSpec((bn, IN), lambda i: (i, 0)),
            pl.BlockSpec((IN, HID), lambda i: (0, 0)),
            pl.BlockSpec((1, HID), lambda i: (0, 0)),
            pl.BlockSpec((1, R, HID, OUT), lambda i: (0, 0, 0, 0)),
        ],
        out_specs=[
            pl.BlockSpec((bn, HID), lambda i: (i, 0)),
            pl.BlockSpec((1, R, bn, OUT), lambda i: (0, 0, i, 0)),
        ],
        out_shape=[
            jax.ShapeDtypeStruct((N, HID), jnp.float32),
            jax.ShapeDtypeStruct((1, R, N, OUT), jnp.float32),
        ],
    )


def _make_combine(n_in: int, n_out: int, relu: bool, bn: int, cat: bool):
    dh = n_out // 2 if cat else n_out
    nb = N // bn

    def body(lo_ref, hi_ref, x_ref, w_ref, b_ref, o_ref):
        m = jnp.dot(x_ref[...], w_ref[...], preferred_element_type=jnp.float32)
        if cat:
            agg = jnp.concatenate([lo_ref[0], hi_ref[0]], axis=-1)
        else:
            agg = lo_ref[0] + hi_ref[0]
        res = agg + m + b_ref[...]
        o_ref[...] = jnp.maximum(res, 0.0) if relu else res
    return pl.pallas_call(
        body,
        grid=(nb,),
        in_specs=[
            pl.BlockSpec((1, bn, dh), lambda i: (0, i, 0)),
            pl.BlockSpec((1, bn, dh), lambda i: (1, i, 0)),
            pl.BlockSpec((bn, n_in), lambda i: (i, 0)),
            pl.BlockSpec((n_in, n_out), lambda i: (0, 0)),
            pl.BlockSpec((1, n_out), lambda i: (0, 0)),
        ],
        out_specs=pl.BlockSpec((bn, n_out), lambda i: (i, 0)),
        out_shape=jax.ShapeDtypeStruct((N, n_out), jnp.float32),
    )


_wcomb1 = _make_wcomb(IN, HID)
_wcomb2 = _make_wcomb(HID, OUT)
_proj1 = _make_proj(IN, HID // 2, 2000, 2)
_combproj = _make_comb_proj(2000)
_edge1 = _make_edge_pass(R * N, HID // 2, edge_split=False)
_edge2 = _make_edge_pass(R * N, OUT, edge_split=True)
_comb2 = _make_combine(HID, OUT, False, 2000, cat=False)


def kernel(x, edge_index, etype, bases1, coeff1, loop1, bias1,
           bases2, coeff2, loop2, bias2):
    w1 = _wcomb1(coeff1, bases1.reshape(B, IN * HID)).reshape(R, IN, HID)
    w1s = jnp.stack([w1[..., :HID // 2], w1[..., HID // 2:]])
    p1 = _proj1(x, w1s)
    src = edge_index[0]
    dst = edge_index[1]
    agg1 = _edge1(p1.reshape(_NC * R * N, HID // 2), src, etype, dst)

    w2 = _wcomb2(coeff2, bases2.reshape(B, HID * OUT)).reshape(1, R, HID, OUT)
    h, p2 = _combproj(agg1, agg1, x, loop1, bias1.reshape(1, HID), w2)
    agg2 = _edge2(p2.reshape(R * N, OUT), src, etype, dst)
    out = _comb2(agg2, agg2, h, loop2, bias2.reshape(1, OUT))
    return out


# speedup vs baseline: 2.9938x; 1.0471x over previous
import functools

import jax
import jax.numpy as jnp
from jax import lax
from jax.experimental import pallas as pl
from jax.experimental.pallas import tpu as pltpu
from jax.experimental.pallas import tpu_sc as plsc

N = 10000
E = 320000
R = 8
B = 8
IN = 128
HID = 256
OUT = 128

_NC = 2
_NS = 16
_L = 16


def _make_edge_pass(n_tab: int, d: int, edge_split: bool):
    K = 80
    n_workers = _NS * _NC if edge_split else _NS
    e_per_w = E // n_workers
    NR = e_per_w // K
    NSB = 5
    SBR = NR // NSB
    SB_E = SBR * K
    ZR = 80
    n_row_chunks = N // ZR
    row_rounds = -(-n_row_chunks // _NS)
    assert e_per_w % K == 0 and N % ZR == 0 and NR % NSB == 0 and SBR > 9

    mesh = plsc.VectorSubcoreMesh(core_axis_name="c", subcore_axis_name="s",
                                  num_cores=_NC, num_subcores=_NS)

    @functools.partial(
        pl.kernel,
        out_type=jax.ShapeDtypeStruct((_NC, N, d), jnp.float32),
        mesh=mesh,
    scratch_types=[
            pltpu.VMEM_SHARED((N, d), jnp.float32),
            pltpu.VMEM((2 * SB_E,), jnp.int32),
            pltpu.VMEM((SB_E,), jnp.int32),
            pltpu.VMEM((K,), jnp.int32),
            pltpu.VMEM((K,), jnp.int32),
            pltpu.VMEM((K,), jnp.int32),
            pltpu.VMEM((K, d), jnp.float32),
            pltpu.VMEM((K, d), jnp.float32),
            pltpu.VMEM((K, d), jnp.float32),
            pltpu.SemaphoreType.DMA,
            pltpu.SemaphoreType.DMA,
            pltpu.SemaphoreType.DMA,
            pltpu.SemaphoreType.DMA,
            pltpu.SemaphoreType.DMA,
            pltpu.SemaphoreType.DMA,
            pltpu.SemaphoreType.DMA,
        ],
    )
    def edge_pass(tab, src, et, dst, out, acc, S, etv, D0, D1, D2,
                  R0, R1, R2, g0, g1, g2, e0, e1, e2, stsem):
        c = lax.axis_index("c")
        s = lax.axis_index("s")

        if edge_split:
            half = 0
            widx = s * _NC + c
        else:
            half = c * n_tab
            widx = s
        w0 = widx * e_per_w

        RB = (R0, R1, R2)
        DB = (D0, D1, D2)
        GS = (g0, g1, g2)
        ES = (e0, e1, e2)

        groups = d // _L

        def _zfill(t, carry):
            R0[t // groups, pl.ds((t % groups) * _L, _L)] = jnp.zeros((_L,), jnp.float32)
            return carry
        lax.fori_loop(0, K * groups, _zfill, 0)

        for q in range(row_rounds):
            chunk_id = s + q * _NS
            @pl.when(chunk_id < n_row_chunks)
            def _():
                pltpu.sync_copy(R0, acc.at[pl.ds(chunk_id * ZR, ZR)])

        def _compute_gidx(soff):
            def _gidx(t, carry):
                sl = pl.ds(soff + t * _L, _L)
                S[sl] = half + etv[pl.ds(t * _L, _L)] * N + S[sl]
                return carry
            lax.fori_loop(0, SB_E // _L, _gidx, 0)

        pltpu.sync_copy(src.at[pl.ds(w0, SB_E)], S.at[pl.ds(0, SB_E)])
        pltpu.sync_copy(et.at[pl.ds(w0, SB_E)], etv)
        _compute_gidx(0)

        def _gather_start(j, i):
            off = (j % (2 * SBR)) * K
            pltpu.async_copy(tab.at[S.at[pl.ds(off, K)]], RB[i], GS[i])

        def _gather_wait(i):
            pltpu.make_async_copy(tab.at[S.at[pl.ds(0, K)]], RB[i], GS[i]).wait()

        def _dst_start(j, i):
            pltpu.async_copy(dst.at[pl.ds(w0 + j * K, K)], DB[i], ES[i])

        def _dst_wait(i):
            pltpu.make_async_copy(dst.at[pl.ds(w0, K)], DB[i], ES[i]).wait()

        for i in range(3):
            _gather_start(i, i)
            _dst_start(i, i)
        plsc.subcore_barrier()

        def _body(u, carry):
            for i in range(3):
                r = 3 * u + i
                _gather_wait(i)
                _dst_wait(i)
                pltpu.sync_copy(RB[i], acc.at[DB[i]], add=True)
                nxt = r + 3

                @pl.when(nxt < NR)
                def _():
                    _gather_start(nxt, i)
                    _dst_start(nxt, i)

                sb1 = r // SBR + 1

                @pl.when(jnp.logical_and(r % SBR == 0, sb1 < NSB))
                def _():
                    soff = (sb1 % 2) * SB_E
                    pltpu.async_copy(src.at[pl.ds(w0 + sb1 * SB_E, SB_E)],
                                     S.at[pl.ds(soff, SB_E)], stsem)
                    pltpu.async_copy(et.at[pl.ds(w0 + sb1 * SB_E, SB_E)],
                                     etv, stsem)

                @pl.when(jnp.logical_and(r % SBR == SBR - 6, sb1 < NSB))
                def _():
                    pltpu.make_async_copy(src.at[pl.ds(w0, SB_E)],
                                          S.at[pl.ds(0, SB_E)], stsem).wait()
                    pltpu.make_async_copy(et.at[pl.ds(w0, SB_E)], etv,
                                          stsem).wait()
                    _compute_gidx((sb1 % 2) * SB_E)
            return carry
        lax.fori_loop(0, NR // 3, _body, 0)
        for i in range(NR % 3):
            _gather_wait(i)
            _dst_wait(i)
            pltpu.sync_copy(RB[i], acc.at[DB[i]], add=True)

        plsc.subcore_barrier()
        for q in range(row_rounds):
            chunk_id = s + q * _NS
            @pl.when(chunk_id < n_row_chunks)
            def _():
                r0 = chunk_id * ZR
                pltpu.sync_copy(acc.at[pl.ds(r0, ZR)], out.at[c, pl.ds(r0, ZR)])

    return edge_pass


def _make_wcomb(m: int):
    def body(c_ref, b_ref, o_ref):
        o_ref[0] = jnp.dot(c_ref[0], b_ref[0],
                           preferred_element_type=jnp.float32)
    return pl.pallas_call(
        body,
        grid=(2,),
        in_specs=[
            pl.BlockSpec((1, R, B), lambda l: (l, 0, 0)),
            pl.BlockSpec((1, B, m), lambda l: (l, 0, 0)),
        ],
        out_specs=pl.BlockSpec((1, R, m), lambda l: (l, 0, 0)),
        out_shape=jax.ShapeDtypeStruct((2, R, m), jnp.float32),
    )


def _make_proj(bn: int):
    nb = N // bn
    dh = HID // 2

    def body(x_ref, w_ref, o_ref):
        t = jnp.dot(x_ref[...], w_ref[0, 0], preferred_element_type=jnp.float32)
        o_ref[0, 0] = t[:, :dh]
        o_ref[1, 0] = t[:, dh:]
    return pl.pallas_call(
        body,
        grid=(nb, R),
        in_specs=[
            pl.BlockSpec((bn, IN), lambda i, r: (i, 0)),
            pl.BlockSpec((1, 1, IN, HID), lambda i, r: (0, r, 0, 0)),
        ],
        out_specs=pl.BlockSpec((2, 1, bn, dh), lambda i, r: (0, r, i, 0)),
        out_shape=jax.ShapeDtypeStruct((2, R, N, dh), jnp.float32),
    )


def _make_comb_proj(bn: int):
    nb = N // bn

    def body(lo_ref, hi_ref, x_ref, w_ref, b_ref, w2_ref, h_ref, p2_ref):
        m = jnp.dot(x_ref[...], w_ref[...], preferred_element_type=jnp.float32)
        agg = jnp.concatenate([lo_ref[0], hi_ref[0]], axis=-1)
        hblk = jnp.maximum(agg + m + b_ref[...], 0.0)
        h_ref[...] = hblk
        for r in range(R):
            p2_ref[0, r] = jnp.dot(hblk, w2_ref[0, r],
                               preferred_element_type=jnp.float32)
    return pl.pallas_call(
        body,
        grid=(nb,),
        in_specs=[
            pl.BlockSpec((1, bn, HID // 2), lambda i: (0, i, 0)),
            pl.BlockSpec((1, bn, HID // 2), lambda i: (1, i, 0)),
            pl.BlockSpec((bn, IN), lambda i: (i, 0)),
            pl.BlockSpec((IN, HID), lambda i: (0, 0)),
            pl.BlockSpec((1, HID), lambda i: (0, 0)),
            pl.BlockSpec((1, R, HID, OUT), lambda i: (1, 0, 0, 0)),
        ],
        out_specs=[
            pl.BlockSpec((bn, HID), lambda i: (i, 0)),
            pl.BlockSpec((1, R, bn, OUT), lambda i: (0, 0, i, 0)),
        ],
        out_shape=[
            jax.ShapeDtypeStruct((N, HID), jnp.float32),
            jax.ShapeDtypeStruct((1, R, N, OUT), jnp.float32),
        ],
    )


def _make_combine(n_in: int, n_out: int, relu: bool, bn: int, cat: bool):
    dh = n_out // 2 if cat else n_out
    nb = N // bn

    def body(lo_ref, hi_ref, x_ref, w_ref, b_ref, o_ref):
        m = jnp.dot(x_ref[...], w_ref[...], preferred_element_type=jnp.float32)
        if cat:
            agg = jnp.concatenate([lo_ref[0], hi_ref[0]], axis=-1)
        else:
            agg = lo_ref[0] + hi_ref[0]
        res = agg + m + b_ref[...]
        o_ref[...] = jnp.maximum(res, 0.0) if relu else res
    return pl.pallas_call(
        body,
        grid=(nb,),
        in_specs=[
            pl.BlockSpec((1, bn, dh), lambda i: (0, i, 0)),
            pl.BlockSpec((1, bn, dh), lambda i: (1, i, 0)),
            pl.BlockSpec((bn, n_in), lambda i: (i, 0)),
            pl.BlockSpec((n_in, n_out), lambda i: (0, 0)),
            pl.BlockSpec((1, n_out), lambda i: (0, 0)),
        ],
        out_specs=pl.BlockSpec((bn, n_out), lambda i: (i, 0)),
        out_shape=jax.ShapeDtypeStruct((N, n_out), jnp.float32),
    )


_wcomb = _make_wcomb(IN * HID)
_proj1 = _make_proj(2000)
_combproj = _make_comb_proj(2000)
_edge1 = _make_edge_pass(R * N, HID // 2, edge_split=False)
_edge2 = _make_edge_pass(R * N, OUT, edge_split=True)
_comb2 = _make_combine(HID, OUT, False, 2000, cat=False)


def kernel(x, edge_index, etype, bases1, coeff1, loop1, bias1,
           bases2, coeff2, loop2, bias2):
    bases_all = jnp.stack([bases1.reshape(B, IN * HID),
                           bases2.reshape(B, HID * OUT)])
    coeff_all = jnp.stack([coeff1, coeff2])
    w_all = _wcomb(coeff_all, bases_all)
    wv1 = w_all.reshape(2, R, IN, HID)
    wv2 = w_all.reshape(2, R, HID, OUT)

    p1 = _proj1(x, wv1)
    src = edge_index[0]
    dst = edge_index[1]
    agg1 = _edge1(p1.reshape(_NC * R * N, HID // 2), src, etype, dst)

    h, p2 = _combproj(agg1, agg1, x, loop1, bias1.reshape(1, HID), wv2)
    agg2 = _edge2(p2.reshape(R * N, OUT), src, etype, dst)
    out = _comb2(agg2, agg2, h, loop2, bias2.reshape(1, OUT))
    return out
